# baseline (speedup 1.0000x reference)
import functools

import jax
import jax.numpy as jnp
from jax import lax
from jax.experimental import pallas as pl
from jax.experimental.pallas import tpu as pltpu

_V = 32
_H = 64
_SBLK = 2048


def _fused_body(ids_ref, lab_ref, embed_ref, w1_ref, b1_ref, w2_ref, b2_ref,
                out_ref, loss_ref, l_scr, logz_scr, c_scr, n_tokens, grid,
                batch):
    i = pl.program_id(0)

    @pl.when(i == 0)
    def _():
        e = embed_ref[...]
        m1 = lax.dot_general(e, w1_ref[...], (((1,), (1,)), ((), ())),
                             preferred_element_type=jnp.float32)
        h = jnp.maximum(m1 + b1_ref[...], 0.0)
        l = lax.dot_general(h, w2_ref[...], (((1,), (1,)), ((), ())),
                            preferred_element_type=jnp.float32)
        l = l + b2_ref[...]
        m = jnp.max(l, axis=1, keepdims=True)
        l_scr[...] = l
        logz_scr[...] = m + jnp.log(jnp.sum(jnp.exp(l - m), axis=1,
                                            keepdims=True))
        c_scr[...] = jnp.zeros((_V, _V), jnp.float32)

    l_tab = l_scr[...]
    iota_s = lax.broadcasted_iota(jnp.int32, (_V, 128), 0)
    c_blk = jnp.zeros((_V, _V), jnp.float32)
    for b in range(batch):
        for g in range(_SBLK // 128):
            sl = pl.ds(g * 128, 128)
            oht = (ids_ref[pl.ds(b, 1), sl] == iota_s).astype(jnp.float32)
            ohlt = (lab_ref[pl.ds(b, 1), sl] == iota_s).astype(jnp.float32)
            out_ref[b, :, sl] = lax.dot_general(
                l_tab, oht, (((0,), (0,)), ((), ())),
                preferred_element_type=jnp.float32)
            c_blk = c_blk + lax.dot_general(
                oht, ohlt, (((1,), (1,)), ((), ())),
                preferred_element_type=jnp.float32)
    c_scr[...] += c_blk

    @pl.when(i == grid - 1)
    def _():
        c = c_scr[...]
        cnt = jnp.sum(c, axis=1, keepdims=True)
        total = jnp.sum(cnt * logz_scr[...]) - jnp.sum(c * l_tab)
        loss_ref[...] = (total / n_tokens).reshape(1, 1)


def kernel(input_ids, labels, embed, fc1_w, fc1_b, fc2_w, fc2_b):
    b, s = input_ids.shape
    n = b * s
    grid = s // _SBLK

    body = functools.partial(_fused_body, n_tokens=float(n), grid=grid,
                             batch=b)
    logits, loss11 = pl.pallas_call(
        body,
        grid=(grid,),
        in_specs=[
            pl.BlockSpec((b, _SBLK), lambda i: (0, i)),
            pl.BlockSpec((b, _SBLK), lambda i: (0, i)),
            pl.BlockSpec((_V, _H), lambda i: (0, 0)),
            pl.BlockSpec((_H, _H), lambda i: (0, 0)),
            pl.BlockSpec((1, _H), lambda i: (0, 0)),
            pl.BlockSpec((_V, _H), lambda i: (0, 0)),
            pl.BlockSpec((1, _V), lambda i: (0, 0)),
        ],
        out_specs=[
            pl.BlockSpec((b, _V, _SBLK), lambda i: (0, 0, i)),
            pl.BlockSpec((1, 1), lambda i: (0, 0)),
        ],
        out_shape=[
            jax.ShapeDtypeStruct((b, _V, s), jnp.float32),
            jax.ShapeDtypeStruct((1, 1), jnp.float32),
        ],
        scratch_shapes=[
            pltpu.VMEM((_V, _V), jnp.float32),
            pltpu.VMEM((_V, 1), jnp.float32),
            pltpu.VMEM((_V, _V), jnp.float32),
        ],
    )(input_ids, labels, embed, fc1_w, fc1_b.reshape(1, _H), fc2_w,
      fc2_b.reshape(1, _V))

    return loss11[0, 0], jnp.transpose(logits, (0, 2, 1))


# speedup vs baseline: 1.0930x; 1.0930x over previous
import functools

import jax
import jax.numpy as jnp
from jax import lax
from jax.experimental import pallas as pl
from jax.experimental.pallas import tpu as pltpu

_V = 32
_H = 64
_SBLK = 2048


def _fused_body(ids_ref, lab_ref, embed_ref, w1_ref, b1_ref, w2_ref, b2_ref,
                out_ref, loss_ref, l_scr, logz_scr, acc_scr, n_tokens, grid,
                batch):
    i = pl.program_id(0)

    @pl.when(i == 0)
    def _():
        e = embed_ref[...]
        m1 = lax.dot_general(e, w1_ref[...], (((1,), (1,)), ((), ())),
                             preferred_element_type=jnp.float32)
        h = jnp.maximum(m1 + b1_ref[...], 0.0)
        l = lax.dot_general(h, w2_ref[...], (((1,), (1,)), ((), ())),
                            preferred_element_type=jnp.float32)
        l = l + b2_ref[...]
        m = jnp.max(l, axis=1, keepdims=True)
        l_scr[...] = l
        logz_scr[...] = m + jnp.log(jnp.sum(jnp.exp(l - m), axis=1,
                                            keepdims=True))
        acc_scr[...] = jnp.zeros((_V, 128), jnp.float32)

    l_tab = l_scr[...]
    logz_b = logz_scr[...]
    iota_s = lax.broadcasted_iota(jnp.int32, (_V, 128), 0)
    acc = jnp.zeros((_V, 128), jnp.float32)
    for b in range(batch):
        for g in range(_SBLK // 128):
            sl = pl.ds(g * 128, 128)
            oht = (ids_ref[pl.ds(b, 1), sl] == iota_s).astype(jnp.float32)
            ohlt = (lab_ref[pl.ds(b, 1), sl] == iota_s).astype(jnp.float32)
            lt_g = lax.dot_general(l_tab, oht, (((0,), (0,)), ((), ())),
                                   preferred_element_type=jnp.float32)
            out_ref[b, :, sl] = lt_g
            acc = acc + (oht * logz_b - lt_g * ohlt)
    acc_scr[...] += acc

    @pl.when(i == grid - 1)
    def _():
        loss_ref[...] = (jnp.sum(acc_scr[...]) / n_tokens).reshape(1, 1)


def kernel(input_ids, labels, embed, fc1_w, fc1_b, fc2_w, fc2_b):
    b, s = input_ids.shape
    n = b * s
    grid = s // _SBLK

    body = functools.partial(_fused_body, n_tokens=float(n), grid=grid,
                             batch=b)
    logits, loss11 = pl.pallas_call(
        body,
        grid=(grid,),
        in_specs=[
            pl.BlockSpec((b, _SBLK), lambda i: (0, i)),
            pl.BlockSpec((b, _SBLK), lambda i: (0, i)),
            pl.BlockSpec((_V, _H), lambda i: (0, 0)),
            pl.BlockSpec((_H, _H), lambda i: (0, 0)),
            pl.BlockSpec((1, _H), lambda i: (0, 0)),
            pl.BlockSpec((_V, _H), lambda i: (0, 0)),
            pl.BlockSpec((1, _V), lambda i: (0, 0)),
        ],
        out_specs=[
            pl.BlockSpec((b, _V, _SBLK), lambda i: (0, 0, i)),
            pl.BlockSpec((1, 1), lambda i: (0, 0)),
        ],
        out_shape=[
            jax.ShapeDtypeStruct((b, _V, s), jnp.float32),
            jax.ShapeDtypeStruct((1, 1), jnp.float32),
        ],
        scratch_shapes=[
            pltpu.VMEM((_V, _V), jnp.float32),
            pltpu.VMEM((_V, 1), jnp.float32),
            pltpu.VMEM((_V, 128), jnp.float32),
        ],
    )(input_ids, labels, embed, fc1_w, fc1_b.reshape(1, _H), fc2_w,
      fc2_b.reshape(1, _V))

    return loss11[0, 0], jnp.transpose(logits, (0, 2, 1))
